# Initial kernel scaffold; baseline (speedup 1.0000x reference)
import jax
import jax.numpy as jnp
from jax.experimental import pallas as pl


def kernel(x, edge_index, t_rand, mask_rand, W):
    raise NotImplementedError("write your pallas kernel here")


# speedup vs baseline: 1.8039x; 1.8039x over previous
import functools

import jax
import jax.numpy as jnp
from jax import lax
from jax.experimental import pallas as pl
from jax.experimental.pallas import tpu as pltpu
from jax.experimental.pallas import tpu_sc as plsc

N = 10000
E = 320000
D = 128
EPSV = 1e-16

NC = 2
NS = 16
NW = NC * NS
LANES = 16
CH = 128
NCHUNK = E // CH
TRASH = 1024
STRIPE = 624
TAIL = N - NS * STRIPE

_NEG_LOG_P_MAX = 27.631021


def _mm_body(x_ref, w_ref, o_ref):
    o_ref[...] = jnp.dot(x_ref[...], w_ref[...],
                         preferred_element_type=jnp.float32)


def _matmul(x, w):
    return pl.pallas_call(
        _mm_body,
        grid=(10,),
        in_specs=[
            pl.BlockSpec((N // 10, D), lambda i: (i, 0)),
            pl.BlockSpec((D, D), lambda i: (0, 0)),
        ],
        out_specs=pl.BlockSpec((N // 10, D), lambda i: (i, 0)),
        out_shape=jax.ShapeDtypeStruct((N, D), jnp.float32),
    )(x, w)


def _comb_body(p0_ref, p1_ref, h_ref, o_ref):
    o_ref[...] = p0_ref[...] + p1_ref[...] - h_ref[...]


def _combine(p0, p1, h):
    spec = pl.BlockSpec((N // 10, D), lambda i: (i, 0))
    return pl.pallas_call(
        _comb_body,
        grid=(10,),
        in_specs=[spec, spec, spec],
        out_specs=spec,
        out_shape=jax.ShapeDtypeStruct((N, D), jnp.float32),
    )(p0, p1, h)


def _sc_scatter_body(h_hbm, src_hbm, dst_hbm, mr_hbm, mc_hbm, part_hbm,
                     accum, srcbuf, dstbuf, mrbuf, dstm, rows, mcbuf, sem):
    cid = lax.axis_index("c")
    sid = lax.axis_index("s")
    w = cid * NS + sid

    pltpu.sync_copy(h_hbm.at[pl.ds(sid * STRIPE, STRIPE)],
                    accum.at[pl.ds(sid * STRIPE, STRIPE)])

    @pl.when(sid == 0)
    def _():
        pltpu.sync_copy(h_hbm.at[pl.ds(NS * STRIPE, TAIL)],
                        accum.at[pl.ds(NS * STRIPE, TAIL)])

    pltpu.sync_copy(mc_hbm, mcbuf)
    plsc.subcore_barrier()

    mc16 = mcbuf[...]
    lane = lax.iota(jnp.int32, LANES)

    def chunk_body(i, _):
        c = i * NW + w

        @pl.when(c < NCHUNK)
        def _():
            base = c * CH
            pltpu.sync_copy(src_hbm.at[pl.ds(base, CH)], srcbuf)
            pltpu.sync_copy(dst_hbm.at[pl.ds(base, CH)], dstbuf)
            pltpu.sync_copy(mr_hbm.at[pl.ds(base, CH)], mrbuf)
            pltpu.async_copy(h_hbm.at[srcbuf], rows, sem).wait()
            for g in range(CH // LANES):
                mr16 = mrbuf[pl.ds(g * LANES, LANES)]
                d16 = dstbuf[pl.ds(g * LANES, LANES)]
                keep = mr16 >= mc16
                toff = (base + g * LANES) % TRASH
                trash16 = N + ((toff + lane) % TRASH)
                dstm[pl.ds(g * LANES, LANES)] = jnp.where(keep, d16, trash16)
            pltpu.sync_copy(rows, accum.at[dstm], add=True)

    lax.fori_loop(0, (NCHUNK + NW - 1) // NW, chunk_body, None)
    plsc.subcore_barrier()
    pltpu.sync_copy(accum.at[pl.ds(sid * STRIPE, STRIPE)],
                    part_hbm.at[cid, pl.ds(sid * STRIPE, STRIPE)])

    @pl.when(sid == 0)
    def _():
        pltpu.sync_copy(accum.at[pl.ds(NS * STRIPE, TAIL)],
                        part_hbm.at[cid, pl.ds(NS * STRIPE, TAIL)])


def _sc_scatter(h, src, dst, mr, mc16):
    mesh = plsc.VectorSubcoreMesh(core_axis_name="c", subcore_axis_name="s")
    f = pl.kernel(
        _sc_scatter_body,
        out_type=jax.ShapeDtypeStruct((NC, N, D), jnp.float32),
        mesh=mesh,
        compiler_params=pltpu.CompilerParams(needs_layout_passes=False),
        scratch_types=[
            pltpu.VMEM_SHARED((N + TRASH, D), jnp.float32),
            pltpu.VMEM((CH,), jnp.int32),
            pltpu.VMEM((CH,), jnp.int32),
            pltpu.VMEM((CH,), jnp.float32),
            pltpu.VMEM((CH,), jnp.int32),
            pltpu.VMEM((CH, D), jnp.float32),
            pltpu.VMEM((LANES,), jnp.float32),
            pltpu.SemaphoreType.DMA,
        ],
    )
    return f(h, src, dst, mr, mc16)


def _softplus_neg(l16):
    u = jnp.exp(-jnp.abs(l16))
    s = u / (2.0 + u)
    s2 = s * s
    log1p_u = s * (2.0 + s2 * (2.0 / 3.0 + s2 * (2.0 / 5.0 + s2 * (2.0 / 7.0))))
    val = jnp.maximum(-l16, 0.0) + log1p_u
    return jnp.minimum(val, _NEG_LOG_P_MAX)


def _sc_loss_body(zt_hbm, src_hbm, dst_hbm, mr_hbm, mc_hbm, out_hbm,
                  srcbuf, dstbuf, mrbuf, rows_a, rows_b, mcbuf, accbuf,
                  sem_a, sem_b):
    cid = lax.axis_index("c")
    sid = lax.axis_index("s")
    w = cid * NS + sid

    pltpu.sync_copy(mc_hbm, mcbuf)
    mc16 = mcbuf[...]
    lane = lax.iota(jnp.int32, LANES)
    zero16 = jnp.zeros((LANES,), jnp.float32)

    def chunk_body(i, acc):
        ci = i * NW + w
        valid = jnp.where(ci < NCHUNK, 1.0, 0.0).astype(jnp.float32)
        c = jnp.minimum(ci, NCHUNK - 1)

        def do_chunk(acc):
            base = c * CH
            pltpu.sync_copy(src_hbm.at[pl.ds(base, CH)], srcbuf)
            pltpu.sync_copy(dst_hbm.at[pl.ds(base, CH)], dstbuf)
            pltpu.sync_copy(mr_hbm.at[pl.ds(base, CH)], mrbuf)
            ca = pltpu.async_copy(zt_hbm.at[srcbuf], rows_a, sem_a)
            cb = pltpu.async_copy(zt_hbm.at[dstbuf], rows_b, sem_b)
            ca.wait()
            cb.wait()

            def group_body(g, acc):
                rowv = g * LANES + lane
                dot = zero16
                for k in range(D):
                    kv = jnp.full((LANES,), k, jnp.int32)
                    va = plsc.load_gather(rows_a, [rowv, kv])
                    vb = plsc.load_gather(rows_b, [rowv, kv])
                    dot = dot + va * vb
                mr16 = plsc.load_gather(mrbuf, [rowv])
                m01 = jnp.where(mr16 < mc16, valid, 0.0)
                return acc + m01 * _softplus_neg(dot)

            return lax.fori_loop(0, CH // LANES, group_body, acc)

        return do_chunk(acc)

    acc = lax.fori_loop(0, (NCHUNK + NW - 1) // NW, chunk_body, zero16)
    accbuf[...] = acc
    pltpu.sync_copy(accbuf, out_hbm.at[w])


def _sc_loss(zt, src, dst, mr, mc16):
    mesh = plsc.VectorSubcoreMesh(core_axis_name="c", subcore_axis_name="s")
    f = pl.kernel(
        _sc_loss_body,
        out_type=jax.ShapeDtypeStruct((NW, LANES), jnp.float32),
        mesh=mesh,
        compiler_params=pltpu.CompilerParams(needs_layout_passes=False),
        scratch_types=[
            pltpu.VMEM((CH,), jnp.int32),
            pltpu.VMEM((CH,), jnp.int32),
            pltpu.VMEM((CH,), jnp.float32),
            pltpu.VMEM((CH, D), jnp.float32),
            pltpu.VMEM((CH, D), jnp.float32),
            pltpu.VMEM((LANES,), jnp.float32),
            pltpu.VMEM((LANES,), jnp.float32),
            pltpu.SemaphoreType.DMA,
            pltpu.SemaphoreType.DMA,
        ],
    )
    return f(zt, src, dst, mr, mc16)


def kernel(x, edge_index, t_rand, mask_rand, W):
    t = (1.0 - EPSV) * t_rand[0] + EPSV
    sigma = -jnp.log1p(-(1.0 - EPSV) * t)
    dsigma = (1.0 - EPSV) / (1.0 - (1.0 - EPSV) * t)
    move_chance = 1.0 - jnp.exp(-sigma)
    coef = dsigma / jnp.expm1(sigma)
    mc16 = jnp.full((LANES,), move_chance, jnp.float32)

    src = edge_index[0].astype(jnp.int32)
    dst = edge_index[1].astype(jnp.int32)
    mr = mask_rand.astype(jnp.float32)

    h = _matmul(x, W)
    part = _sc_scatter(h, src, dst, mr, mc16)
    zt = _combine(part[0], part[1], h)
    partials = _sc_loss(zt, src, dst, mr, mc16)
    return coef * jnp.sum(partials)
